# baseline (speedup 1.0000x reference)
import functools

import jax
import jax.numpy as jnp
from jax import lax
from jax.experimental import pallas as pl
from jax.experimental.pallas import tpu as pltpu
from jax.experimental.pallas import tpu_sc as plsc

VOCAB = 1000000
DIM = 300
B = 4096
L = 50

NC = 2
NS = 16
NW = NC * NS
ROWS = 3 * B
RPW = ROWS // NW
NCH = 19
DPAD = NCH * 16
GW = 16
NG = 20
NGR = VOCAB * DIM // GW
GPR = L * NG
NT = 8
TSZ = GPR // NT


def _compute_row(buf, sst, orow):
    iota = lax.iota(jnp.int32, 16)
    ninf = jnp.full((16,), -jnp.inf, jnp.float32)
    accs = tuple(ninf for _ in range(NCH))

    def mbody(m, accs):
        sm = plsc.load_gather(sst, [jnp.full((16,), m, jnp.int32)])
        base20 = m * NG
        pv = sm + iota
        d1 = pv & 15
        d0b = (pv >> 4) + base20
        accs = list(accs)
        for c in range(NCH - 1):
            v = plsc.load_gather(buf, [d0b + c, d1])
            accs[c] = jnp.maximum(accs[c], v)
        pt = sm + (DIM - 16) + iota
        v = plsc.load_gather(buf, [(pt >> 4) + base20, pt & 15])
        accs[NCH - 1] = jnp.maximum(accs[NCH - 1], v)
        return tuple(accs)

    accs = lax.fori_loop(0, L, mbody, accs)
    for c in range(NCH):
        orow[c] = accs[c]


def _sc_gather_maxpool(embg, gidx, svals):
    mesh = plsc.VectorSubcoreMesh(core_axis_name="c", subcore_axis_name="s")

    @functools.partial(
        pl.kernel,
        mesh=mesh,
        compiler_params=pltpu.CompilerParams(
            use_tc_tiling_on_sc=False, needs_layout_passes=False),
        out_type=jax.ShapeDtypeStruct((ROWS, NCH, 16), jnp.float32),
        scratch_types=[
            pltpu.VMEM((NT, TSZ), jnp.int32),
            pltpu.VMEM((NT, TSZ), jnp.int32),
            pltpu.VMEM((64,), jnp.int32),
            pltpu.VMEM((64,), jnp.int32),
            pltpu.VMEM((GPR, GW), jnp.float32),
            pltpu.VMEM((GPR, GW), jnp.float32),
            pltpu.VMEM((NCH, 16), jnp.float32),
            pltpu.SemaphoreType.DMA,
            pltpu.SemaphoreType.DMA,
        ],
    )
    def sc_kernel(emb_hbm, gidx_hbm, s_hbm, out_hbm, gst_a, gst_b,
                  sst_a, sst_b, buf_a, buf_b, orow, sem_a, sem_b):
        wid = lax.axis_index("s") * NC + lax.axis_index("c")
        base = wid * RPW

        def issue(r, gst, sst, buf, sem):
            pltpu.sync_copy(gidx_hbm.at[r], gst)
            pltpu.sync_copy(s_hbm.at[r], sst)
            return [
                pltpu.async_copy(emb_hbm.at[gst.at[t]],
                                 buf.at[pl.ds(t * TSZ, TSZ)], sem)
                for t in range(NT)
            ]

        def body(i, carry):
            r0 = base + 2 * i
            r1 = base + 2 * i + 1
            cps_a = issue(r0, gst_a, sst_a, buf_a, sem_a)
            cps_b = issue(r1, gst_b, sst_b, buf_b, sem_b)
            for cp in cps_a:
                cp.wait()
            _compute_row(buf_a, sst_a, orow)
            pltpu.sync_copy(orow, out_hbm.at[r0])
            for cp in cps_b:
                cp.wait()
            _compute_row(buf_b, sst_b, orow)
            pltpu.sync_copy(orow, out_hbm.at[r1])
            return carry

        lax.fori_loop(0, RPW // 2, body, 0)

    return sc_kernel(embg, gidx, svals)


def _tc_head(pooled, w2, b2, lab2):
    def head_body(p_ref, w_ref, b_ref, lab_ref, loss_ref, logits_ref):
        r = p_ref[0:B, :]
        w0 = p_ref[B:2 * B, :]
        w1 = p_ref[2 * B:3 * B, :]
        wr = w_ref[0:1, :]
        ww = w_ref[1:2, :]
        dr = jnp.sum(r * wr, axis=1, keepdims=True)
        d0 = jnp.sum(w0 * ww, axis=1, keepdims=True)
        d1 = jnp.sum(w1 * ww, axis=1, keepdims=True)
        bb = b_ref[0, 0]
        l0 = dr + d0 + bb
        l1 = dr + d1 + bb
        logits = jnp.concatenate([l0, l1], axis=1)
        logits_ref[...] = logits
        m = jnp.maximum(l0, l1)
        lse = m + jnp.log(jnp.exp(l0 - m) + jnp.exp(l1 - m))
        lab = lab_ref[...]
        lsel = jnp.where(lab == 0, l0, l1)
        loss_ref[...] = jnp.sum(lse - lsel).reshape(1, 1) / B

    return pl.pallas_call(
        head_body,
        out_shape=[
            jax.ShapeDtypeStruct((1, 1), jnp.float32),
            jax.ShapeDtypeStruct((B, 2), jnp.float32),
        ],
    )(pooled, w2, b2, lab2)


def kernel(reasons, warrant0s, warrant1s, label_ids, emb, W, b):
    idx = jnp.concatenate([reasons, warrant0s, warrant1s], axis=0).astype(jnp.int32)
    embg = emb.reshape(NGR, GW)
    g0 = (idx * DIM) // GW
    gidx = jnp.minimum(g0[:, :, None] + jnp.arange(NG, dtype=jnp.int32),
                       NGR - 1).reshape(ROWS, NT, TSZ)
    svals = jnp.concatenate(
        [(idx * 12) & 15,
         jnp.zeros((ROWS, 64 - L), jnp.int32)], axis=1)
    pooled = _sc_gather_maxpool(embg, gidx, svals).reshape(ROWS, DPAD)
    w2t = W[:, 0].reshape(2, DIM)
    w2 = jnp.concatenate(
        [w2t[:, :288], jnp.zeros((2, 4), jnp.float32), w2t[:, 288:]], axis=1)
    b2 = b.reshape(1, 1)
    lab2 = label_ids.astype(jnp.int32).reshape(B, 1)
    loss, logits = _tc_head(pooled, w2, b2, lab2)
    return (loss[0, 0], logits)


# speedup vs baseline: 3.3879x; 3.3879x over previous
import functools

import jax
import jax.numpy as jnp
from jax import lax
from jax.experimental import pallas as pl
from jax.experimental.pallas import tpu as pltpu
from jax.experimental.pallas import tpu_sc as plsc

VOCAB = 1000000
DIM = 300
B = 4096
L = 50

NC = 2
NS = 16
NW = NC * NS
ROWS = 3 * B
RPW = ROWS // NW
NCH = 19
DPAD = NCH * 16
NSEG = 2
TOFF = NSEG * 128
TW = DIM - TOFF
TAIL_OFFS = (0, 16, 28)
GPP = 2


def _compute_pair(bufsegs, bt, p, orow):
    base = p * L

    def loads(l):
        vals = []
        for c in range(NSEG * 8):
            k, off = divmod(c * 16, 128)
            vals.append(bufsegs[k][base + l, pl.ds(off, 16)])
        for toff in TAIL_OFFS:
            vals.append(bt[l, pl.ds(toff, 16)])
        return vals

    def lbody(l, accs):
        return tuple(jnp.maximum(a, v) for a, v in zip(accs, loads(l)))

    accs = lax.fori_loop(1, L, lbody, tuple(loads(0)))
    for c in range(NCH):
        orow[pl.ds(c * 16, 16)] = accs[c]


def _sc_gather_maxpool(emb, idx64):
    mesh = plsc.VectorSubcoreMesh(core_axis_name="c", subcore_axis_name="s")

    @functools.partial(
        pl.kernel,
        mesh=mesh,
        out_type=jax.ShapeDtypeStruct((ROWS * DPAD,), jnp.float32),
        scratch_types=[
            pltpu.VMEM((GPP * 64,), jnp.int32),
            pltpu.VMEM((GPP * 64,), jnp.int32),
            [pltpu.VMEM((GPP * L, 128), jnp.float32) for _ in range(NSEG)],
            [pltpu.VMEM((GPP * L, 128), jnp.float32) for _ in range(NSEG)],
            [pltpu.VMEM((L, TW), jnp.float32) for _ in range(GPP)],
            [pltpu.VMEM((L, TW), jnp.float32) for _ in range(GPP)],
            [pltpu.VMEM((DPAD,), jnp.float32) for _ in range(2 * GPP)],
            pltpu.SemaphoreType.DMA,
            pltpu.SemaphoreType.DMA,
            pltpu.SemaphoreType.DMA,
        ],
    )
    def sc_kernel(emb_hbm, idx_hbm, out_hbm, idxg_a, idxg_b,
                  segs_a, segs_b, bts_a, bts_b, orows, sem_a, sem_b, sem_o):
        wid = lax.axis_index("s") * NC + lax.axis_index("c")
        base = wid * RPW

        def issue(r0, idxg, segs, bts, sem):
            pltpu.sync_copy(idx_hbm.at[pl.ds(r0 * 64, GPP * 64)], idxg)
            cps = []
            for p in range(GPP):
                for k in range(NSEG):
                    cps.append(pltpu.async_copy(
                        emb_hbm.at[idxg.at[pl.ds(p * 64, L)],
                                   pl.ds(k * 128, 128)],
                        segs[k].at[pl.ds(p * L, L)], sem))
            for p in range(GPP):
                for mc in range((L + 15) // 16):
                    iv = idxg[pl.ds(p * 64 + mc * 16, 16)]
                    for mi in range(min(16, L - mc * 16)):
                        m = mc * 16 + mi
                        cps.append(pltpu.async_copy(
                            emb_hbm.at[iv[mi], pl.ds(TOFF, TW)],
                            bts[p].at[m], sem))
            return cps

        def group_compute(r0, segs, bts, oro):
            cpos = []
            for p in range(GPP):
                _compute_pair(segs, bts[p], p, oro[p])
                cpos.append(pltpu.async_copy(
                    oro[p], out_hbm.at[pl.ds((r0 + p) * DPAD, DPAD)], sem_o))
            return cpos

        def body(i, carry):
            ra = base + 2 * GPP * i
            rb = ra + GPP
            cps_a = issue(ra, idxg_a, segs_a, bts_a, sem_a)
            cps_b = issue(rb, idxg_b, segs_b, bts_b, sem_b)
            for cp in cps_a:
                cp.wait()
            cpos = group_compute(ra, segs_a, bts_a, orows[:GPP])
            for cp in cps_b:
                cp.wait()
            cpos += group_compute(rb, segs_b, bts_b, orows[GPP:])
            for cp in cpos:
                cp.wait()
            return carry

        lax.fori_loop(0, RPW // (2 * GPP), body, 0)

    return sc_kernel(emb, idx64)


def kernel(reasons, warrant0s, warrant1s, label_ids, emb, W, b):
    idx = jnp.concatenate([reasons, warrant0s, warrant1s], axis=0).astype(jnp.int32)
    idx64 = jnp.concatenate(
        [idx, jnp.zeros((ROWS, 64 - L), jnp.int32)], axis=1).reshape(-1)
    pooled = _sc_gather_maxpool(emb, idx64).reshape(ROWS, DPAD)
    def true300(block):
        return jnp.concatenate([block[:, :288], block[:, 292:]], axis=1)

    r = true300(pooled[0:B])
    w0 = true300(pooled[B:2 * B])
    w1 = true300(pooled[2 * B:3 * B])
    input0 = jnp.concatenate([r, w0], axis=1)
    input1 = jnp.concatenate([r, w1], axis=1)
    logits0 = input0 @ W + b
    logits1 = input1 @ W + b
    logits = jnp.concatenate([logits0, logits1], axis=1)
    logp = jax.nn.log_softmax(logits, axis=1)
    nll = -jnp.take_along_axis(logp, label_ids[:, None], axis=1)[:, 0]
    loss = jnp.mean(nll)
    return (loss, logits)
